# baseline (speedup 1.0000x reference)
import functools

import jax
import jax.numpy as jnp
from jax import lax
from jax.experimental import pallas as pl
from jax.experimental.pallas import tpu as pltpu, tpu_sc as plsc

N_TOKENS = 32768
DIM = 2048
NUM_EXPERTS = 8
TOP_K = 2
BLOCK_T = 2048

_SC_INFO = plsc.get_sparse_core_info()
_NC, _NS, _L = _SC_INFO.num_cores, _SC_INFO.num_subcores, _SC_INFO.num_lanes
_NW = _NC * _NS

SC_TOKENS = 4096
TC_TOKENS = N_TOKENS - SC_TOKENS
_TOK_PER_W = TC_TOKENS // _NW
_TAIL_PER_W = SC_TOKENS // _NW
_TAIL_CHUNK = 16
_KCH = DIM // _L


def _gate_kernel(x_ref, wt_ref, scores_ref):
    logits = jnp.dot(x_ref[...], wt_ref[...], preferred_element_type=jnp.float32)
    scores_ref[...] = jax.nn.sigmoid(logits).T


def _gate_scores_t(x, gate_w_t):
    return pl.pallas_call(
        _gate_kernel,
        grid=(TC_TOKENS // BLOCK_T,),
        in_specs=[
            pl.BlockSpec((BLOCK_T, DIM), lambda i: (i, 0)),
            pl.BlockSpec((DIM, NUM_EXPERTS), lambda i: (0, 0)),
        ],
        out_specs=pl.BlockSpec((NUM_EXPERTS, BLOCK_T), lambda i: (0, i)),
        out_shape=jax.ShapeDtypeStruct((NUM_EXPERTS, TC_TOKENS), jnp.float32),
    )(x, gate_w_t)


def _round_bf16(v):
    b = lax.bitcast_convert_type(v, jnp.int32)
    lsb = lax.shift_right_logical(b, 16) & 1
    b = b + 0x7FFF + lsb
    b = b & jnp.int32(-65536)
    return lax.bitcast_convert_type(b, jnp.float32)


def _soft_sigmoid(l):
    LOG2E = 1.4426950408889634
    LN2 = 0.6931471805599453
    z = l * (-LOG2E)
    z = jnp.minimum(jnp.maximum(z, -126.0), 126.0)
    i = z.astype(jnp.int32)
    f = z - i.astype(jnp.float32)
    neg = f < 0
    i = jnp.where(neg, i - 1, i)
    f = jnp.where(neg, f + 1.0, f)
    u = f * LN2
    p = jnp.full_like(u, 1.0 / 40320.0)
    for c in (1.0 / 5040, 1.0 / 720, 1.0 / 120, 1.0 / 24, 1.0 / 6, 0.5, 1.0, 1.0):
        p = p * u + c
    e2n = lax.bitcast_convert_type((i + 127) << 23, jnp.float32)
    return 1.0 / (1.0 + e2n * p)


def _perm_gather(v, idx):
    return lax.gather(
        v, idx[:, None],
        dimension_numbers=lax.GatherDimensionNumbers(
            offset_dims=(), collapsed_slice_dims=(0,), start_index_map=(0,)),
        slice_sizes=(1,),
        mode=lax.GatherScatterMode.PROMISE_IN_BOUNDS)


def _route_body(scores_hbm, bias_hbm, s1_hbm, s2_hbm, i1_hbm, i2_hbm, pcnt_hbm,
                sc_v, bias_v, s1_v, s2_v, i1_v, i2_v, cnt_v):
    wid = lax.axis_index("s") * _NC + lax.axis_index("c")
    base = wid * _TOK_PER_W
    pltpu.sync_copy(scores_hbm.at[:, pl.ds(base, _TOK_PER_W)], sc_v)
    pltpu.sync_copy(bias_hbm, bias_v)

    bias_regs = [bias_v[e, :] for e in range(NUM_EXPERTS)]
    zero = jnp.zeros((_L,), jnp.float32)

    def chunk(j, cnts):
        t = j * _L
        s0 = sc_v[0, pl.ds(t, _L)]
        r1 = s0 + bias_regs[0]
        g1 = s0
        i1 = jnp.zeros((_L,), jnp.int32)
        r2 = jnp.full((_L,), -jnp.inf, jnp.float32)
        g2 = zero
        i2 = jnp.zeros((_L,), jnp.int32)
        for e in range(1, NUM_EXPERTS):
            s = sc_v[e, pl.ds(t, _L)]
            r = s + bias_regs[e]
            ei = jnp.full((_L,), e, jnp.int32)
            gt1 = r > r1
            gt2 = r > r2
            r2 = jnp.where(gt1, r1, jnp.where(gt2, r, r2))
            g2 = jnp.where(gt1, g1, jnp.where(gt2, s, g2))
            i2 = jnp.where(gt1, i1, jnp.where(gt2, ei, i2))
            r1 = jnp.where(gt1, r, r1)
            g1 = jnp.where(gt1, s, g1)
            i1 = jnp.where(gt1, ei, i1)
        rden = 1.0 / (g1 + g2 + 1e-20)
        s1_v[pl.ds(t, _L)] = g1 * rden
        s2_v[pl.ds(t, _L)] = g2 * rden
        i1_v[pl.ds(t, _L)] = i1
        i2_v[pl.ds(t, _L)] = i2
        new = []
        for e in range(NUM_EXPERTS):
            hits = (jnp.where(i1 == e, 1.0, 0.0) + jnp.where(i2 == e, 1.0, 0.0))
            new.append(cnts[e] + hits)
        return tuple(new)

    cnts = lax.fori_loop(0, _TOK_PER_W // _L, chunk,
                         tuple(zero for _ in range(NUM_EXPERTS)))

    lane = lax.iota(jnp.int32, _L)
    total = jnp.zeros((_L,), jnp.float32)
    for e in range(NUM_EXPERTS):
        ce = cnts[e]
        for k in (1, 2, 4, 8):
            ce = ce + _perm_gather(ce, jnp.bitwise_xor(lane, k))
        total = total + jnp.where(lane == e, ce, 0.0)
    cnt_v[...] = total

    pltpu.sync_copy(s1_v, s1_hbm.at[pl.ds(base, _TOK_PER_W)])
    pltpu.sync_copy(s2_v, s2_hbm.at[pl.ds(base, _TOK_PER_W)])
    pltpu.sync_copy(i1_v, i1_hbm.at[pl.ds(base, _TOK_PER_W)])
    pltpu.sync_copy(i2_v, i2_hbm.at[pl.ds(base, _TOK_PER_W)])
    pltpu.sync_copy(cnt_v, pcnt_hbm.at[wid])


_route = functools.partial(
    pl.kernel,
    mesh=plsc.VectorSubcoreMesh(core_axis_name="c", subcore_axis_name="s"),
    out_type=[
        jax.ShapeDtypeStruct((TC_TOKENS,), jnp.float32),
        jax.ShapeDtypeStruct((TC_TOKENS,), jnp.float32),
        jax.ShapeDtypeStruct((TC_TOKENS,), jnp.int32),
        jax.ShapeDtypeStruct((TC_TOKENS,), jnp.int32),
        jax.ShapeDtypeStruct((_NW, _L), jnp.float32),
    ],
    scratch_types=[
        pltpu.VMEM((NUM_EXPERTS, _TOK_PER_W), jnp.float32),
        pltpu.VMEM((NUM_EXPERTS, _L), jnp.float32),
        pltpu.VMEM((_TOK_PER_W,), jnp.float32),
        pltpu.VMEM((_TOK_PER_W,), jnp.float32),
        pltpu.VMEM((_TOK_PER_W,), jnp.int32),
        pltpu.VMEM((_TOK_PER_W,), jnp.int32),
        pltpu.VMEM((_L,), jnp.float32),
    ],
)(_route_body)


def _tail_body(x_hbm, w_hbm, bias_hbm, s1_hbm, s2_hbm, i1_hbm, i2_hbm, pcnt_hbm,
               xb_v, w_v, bias_v, s1_v, s2_v, i1_v, i2_v, cnt_v):
    wid = lax.axis_index("s") * _NC + lax.axis_index("c")
    row_base = TC_TOKENS + wid * _TAIL_PER_W
    pltpu.sync_copy(w_hbm, w_v)
    pltpu.sync_copy(bias_hbm, bias_v)
    bias_vec = bias_v[...]
    lane = lax.iota(jnp.int32, _L)
    perms = {k: jnp.bitwise_xor(lane, k) for k in (1, 2, 4, 8)}
    zero = jnp.zeros((_L,), jnp.float32)
    big_i = jnp.full((_L,), _L, jnp.int32)

    def allsum(v):
        for k in (1, 2, 4, 8):
            v = v + _perm_gather(v, perms[k])
        return v

    def allmax(v):
        for k in (1, 2, 4, 8):
            v = jnp.maximum(v, _perm_gather(v, perms[k]))
        return v

    def allmin_i(v):
        for k in (1, 2, 4, 8):
            v = jnp.minimum(v, _perm_gather(v, perms[k]))
        return v

    def chunk_body(c, cnt_vec):
        t0 = c * _TAIL_CHUNK
        pltpu.sync_copy(x_hbm.at[pl.ds(row_base + t0, _TAIL_CHUNK)], xb_v)
        out_s1, out_s2 = zero, zero
        out_i1 = jnp.zeros((_L,), jnp.int32)
        out_i2 = jnp.zeros((_L,), jnp.int32)
        for g in range(_TAIL_CHUNK // 4):
            def kbody(kc, accs):
                kb = kc * _L
                xs = [_round_bf16(xb_v[g * 4 + t, pl.ds(kb, _L)])
                      for t in range(4)]
                new = list(accs)
                for e in range(NUM_EXPERTS):
                    wv = w_v[e, pl.ds(kb, _L)]
                    for t in range(4):
                        new[t * NUM_EXPERTS + e] = (
                            new[t * NUM_EXPERTS + e] + xs[t] * wv)
                return tuple(new)

            accs = lax.fori_loop(0, _KCH, kbody,
                                 tuple(zero for _ in range(4 * NUM_EXPERTS)))
            for t in range(4):
                tok = g * 4 + t
                logit = zero
                for e in range(NUM_EXPERTS):
                    v = allsum(accs[t * NUM_EXPERTS + e])
                    logit = logit + jnp.where(lane == e, v, 0.0)
                score = _soft_sigmoid(logit)
                routing = score + bias_vec
                m1 = allmax(routing)
                i1 = allmin_i(jnp.where(routing == m1, lane, big_i))
                masked = jnp.where(lane == i1, -jnp.inf, routing)
                m2 = allmax(masked)
                i2 = allmin_i(jnp.where(masked == m2, lane, big_i))
                g1 = _perm_gather(score, i1)
                g2 = _perm_gather(score, i2)
                rden = 1.0 / (g1 + g2 + 1e-20)
                sel = lane == tok
                out_s1 = out_s1 + jnp.where(sel, g1 * rden, 0.0)
                out_s2 = out_s2 + jnp.where(sel, g2 * rden, 0.0)
                out_i1 = out_i1 + jnp.where(sel, i1, 0)
                out_i2 = out_i2 + jnp.where(sel, i2, 0)
                cnt_vec = (cnt_vec + jnp.where(lane == i1, 1.0, 0.0)
                           + jnp.where(lane == i2, 1.0, 0.0))
        s1_v[pl.ds(t0, _TAIL_CHUNK)] = out_s1
        s2_v[pl.ds(t0, _TAIL_CHUNK)] = out_s2
        i1_v[pl.ds(t0, _TAIL_CHUNK)] = out_i1
        i2_v[pl.ds(t0, _TAIL_CHUNK)] = out_i2
        return cnt_vec

    cnt = lax.fori_loop(0, _TAIL_PER_W // _TAIL_CHUNK, chunk_body, zero)
    cnt_v[...] = cnt

    base = wid * _TAIL_PER_W
    pltpu.sync_copy(s1_v, s1_hbm.at[pl.ds(base, _TAIL_PER_W)])
    pltpu.sync_copy(s2_v, s2_hbm.at[pl.ds(base, _TAIL_PER_W)])
    pltpu.sync_copy(i1_v, i1_hbm.at[pl.ds(base, _TAIL_PER_W)])
    pltpu.sync_copy(i2_v, i2_hbm.at[pl.ds(base, _TAIL_PER_W)])
    pltpu.sync_copy(cnt_v, pcnt_hbm.at[wid])


_tail = functools.partial(
    pl.kernel,
    mesh=plsc.VectorSubcoreMesh(core_axis_name="c", subcore_axis_name="s"),
    out_type=[
        jax.ShapeDtypeStruct((SC_TOKENS,), jnp.float32),
        jax.ShapeDtypeStruct((SC_TOKENS,), jnp.float32),
        jax.ShapeDtypeStruct((SC_TOKENS,), jnp.int32),
        jax.ShapeDtypeStruct((SC_TOKENS,), jnp.int32),
        jax.ShapeDtypeStruct((_NW, _L), jnp.float32),
    ],
    scratch_types=[
        pltpu.VMEM((_TAIL_CHUNK, DIM), jnp.float32),
        pltpu.VMEM((NUM_EXPERTS, DIM), jnp.float32),
        pltpu.VMEM((_L,), jnp.float32),
        pltpu.VMEM((_TAIL_PER_W,), jnp.float32),
        pltpu.VMEM((_TAIL_PER_W,), jnp.float32),
        pltpu.VMEM((_TAIL_PER_W,), jnp.int32),
        pltpu.VMEM((_TAIL_PER_W,), jnp.int32),
        pltpu.VMEM((_L,), jnp.float32),
    ],
)(_tail_body)


@jax.jit
def kernel(x, expert_bias, gate_w):
    bias_lane = jnp.concatenate(
        [expert_bias, jnp.full((_L - NUM_EXPERTS,), -jnp.inf, jnp.float32)])
    gate_w_rnd = _round_bf16(gate_w)
    ts1, ts2, ti1, ti2, tpcnt = _tail(x, gate_w_rnd, bias_lane)

    scores_t = _gate_scores_t(x, gate_w.T)
    bias_b = jnp.broadcast_to(expert_bias[:, None], (NUM_EXPERTS, _L))
    s1, s2, i1, i2, pcnt = _route(scores_t, bias_b)

    top_scores = jnp.concatenate(
        [jnp.stack([s1, s2], axis=1), jnp.stack([ts1, ts2], axis=1)], axis=0)
    idx = jnp.concatenate(
        [jnp.stack([i1, i2], axis=1), jnp.stack([ti1, ti2], axis=1)],
        axis=0).astype(jnp.int64)
    counts = (jnp.sum(pcnt, axis=0) + jnp.sum(tpcnt, axis=0))[:NUM_EXPERTS]
    return top_scores, idx, counts


# speedup vs baseline: 1.7566x; 1.7566x over previous
import functools

import jax
import jax.numpy as jnp
from jax import lax
from jax.experimental import pallas as pl
from jax.experimental.pallas import tpu as pltpu, tpu_sc as plsc

N_TOKENS = 32768
DIM = 2048
NUM_EXPERTS = 8
TOP_K = 2
BLOCK_T = 2048
HALF_T = N_TOKENS // 2

_SC_INFO = plsc.get_sparse_core_info()
_NC, _NS, _L = _SC_INFO.num_cores, _SC_INFO.num_subcores, _SC_INFO.num_lanes
_NW = _NC * _NS
_TOK_PER_W = HALF_T // _NW


def _gate_kernel(x_ref, wt_ref, scores_ref):
    logits = jnp.dot(x_ref[...], wt_ref[...], preferred_element_type=jnp.float32)
    scores_ref[...] = jax.nn.sigmoid(logits).T


def _gate_scores_t(x, gate_w_t, half):
    off = half * (HALF_T // BLOCK_T)
    return pl.pallas_call(
        _gate_kernel,
        grid=(HALF_T // BLOCK_T,),
        in_specs=[
            pl.BlockSpec((BLOCK_T, DIM), lambda i: (i + off, 0)),
            pl.BlockSpec((DIM, NUM_EXPERTS), lambda i: (0, 0)),
        ],
        out_specs=pl.BlockSpec((NUM_EXPERTS, BLOCK_T), lambda i: (0, i)),
        out_shape=jax.ShapeDtypeStruct((NUM_EXPERTS, HALF_T), jnp.float32),
    )(x, gate_w_t)


def _route_body(scores_hbm, bias_hbm, s1_hbm, s2_hbm, i1_hbm, i2_hbm, pcnt_hbm,
                sc_v, bias_v, s1_v, s2_v, i1_v, i2_v, cnt_v):
    wid = lax.axis_index("s") * _NC + lax.axis_index("c")
    base = wid * _TOK_PER_W
    pltpu.sync_copy(scores_hbm.at[:, pl.ds(base, _TOK_PER_W)], sc_v)
    pltpu.sync_copy(bias_hbm, bias_v)

    bias_regs = [bias_v[e, :] for e in range(NUM_EXPERTS)]
    zero = jnp.zeros((_L,), jnp.float32)

    def chunk(j, cnts):
        t = j * _L
        s0 = sc_v[0, pl.ds(t, _L)]
        r1 = s0 + bias_regs[0]
        g1 = s0
        i1 = jnp.zeros((_L,), jnp.int32)
        r2 = jnp.full((_L,), -jnp.inf, jnp.float32)
        g2 = zero
        i2 = jnp.zeros((_L,), jnp.int32)
        for e in range(1, NUM_EXPERTS):
            s = sc_v[e, pl.ds(t, _L)]
            r = s + bias_regs[e]
            ei = jnp.full((_L,), e, jnp.int32)
            gt1 = r > r1
            gt2 = r > r2
            r2 = jnp.where(gt1, r1, jnp.where(gt2, r, r2))
            g2 = jnp.where(gt1, g1, jnp.where(gt2, s, g2))
            i2 = jnp.where(gt1, i1, jnp.where(gt2, ei, i2))
            r1 = jnp.where(gt1, r, r1)
            g1 = jnp.where(gt1, s, g1)
            i1 = jnp.where(gt1, ei, i1)
        denom = g1 + g2 + 1e-20
        s1_v[pl.ds(t, _L)] = g1 / denom
        s2_v[pl.ds(t, _L)] = g2 / denom
        i1_v[pl.ds(t, _L)] = i1
        i2_v[pl.ds(t, _L)] = i2
        new = []
        for e in range(NUM_EXPERTS):
            hits = (jnp.where(i1 == e, 1.0, 0.0) + jnp.where(i2 == e, 1.0, 0.0))
            new.append(cnts[e] + hits)
        return tuple(new)

    cnts = lax.fori_loop(0, _TOK_PER_W // _L, chunk,
                         tuple(zero for _ in range(NUM_EXPERTS)))

    lane = lax.iota(jnp.int32, _L)
    total = jnp.zeros((_L,), jnp.float32)
    for e in range(NUM_EXPERTS):
        ce = cnts[e]
        for k in (1, 2, 4, 8):
            perm = jnp.bitwise_xor(lane, k)
            ce = ce + lax.gather(
                ce, perm[:, None],
                dimension_numbers=lax.GatherDimensionNumbers(
                    offset_dims=(), collapsed_slice_dims=(0,),
                    start_index_map=(0,)),
                slice_sizes=(1,),
                mode=lax.GatherScatterMode.PROMISE_IN_BOUNDS)
        total = total + jnp.where(lane == e, ce, 0.0)
    cnt_v[...] = total

    pltpu.sync_copy(s1_v, s1_hbm.at[pl.ds(base, _TOK_PER_W)])
    pltpu.sync_copy(s2_v, s2_hbm.at[pl.ds(base, _TOK_PER_W)])
    pltpu.sync_copy(i1_v, i1_hbm.at[pl.ds(base, _TOK_PER_W)])
    pltpu.sync_copy(i2_v, i2_hbm.at[pl.ds(base, _TOK_PER_W)])
    pltpu.sync_copy(cnt_v, pcnt_hbm.at[wid])


_route = functools.partial(
    pl.kernel,
    mesh=plsc.VectorSubcoreMesh(core_axis_name="c", subcore_axis_name="s"),
    out_type=[
        jax.ShapeDtypeStruct((HALF_T,), jnp.float32),
        jax.ShapeDtypeStruct((HALF_T,), jnp.float32),
        jax.ShapeDtypeStruct((HALF_T,), jnp.int32),
        jax.ShapeDtypeStruct((HALF_T,), jnp.int32),
        jax.ShapeDtypeStruct((_NW, _L), jnp.float32),
    ],
    scratch_types=[
        pltpu.VMEM((NUM_EXPERTS, _TOK_PER_W), jnp.float32),
        pltpu.VMEM((NUM_EXPERTS, _L), jnp.float32),
        pltpu.VMEM((_TOK_PER_W,), jnp.float32),
        pltpu.VMEM((_TOK_PER_W,), jnp.float32),
        pltpu.VMEM((_TOK_PER_W,), jnp.int32),
        pltpu.VMEM((_TOK_PER_W,), jnp.int32),
        pltpu.VMEM((_L,), jnp.float32),
    ],
)(_route_body)


@jax.jit
def kernel(x, expert_bias, gate_w):
    gate_w_t = gate_w.T
    bias_b = jnp.broadcast_to(expert_bias[:, None], (NUM_EXPERTS, _L))
    scores_a = _gate_scores_t(x, gate_w_t, 0)
    ra = _route(scores_a, bias_b)
    scores_b = _gate_scores_t(x, gate_w_t, 1)
    rb = _route(scores_b, bias_b)
    s1 = jnp.concatenate([ra[0], rb[0]])
    s2 = jnp.concatenate([ra[1], rb[1]])
    i1 = jnp.concatenate([ra[2], rb[2]])
    i2 = jnp.concatenate([ra[3], rb[3]])
    top_scores = jnp.stack([s1, s2], axis=1)
    idx = jnp.stack([i1, i2], axis=1).astype(jnp.int64)
    counts = (jnp.sum(ra[4], axis=0) + jnp.sum(rb[4], axis=0))[:NUM_EXPERTS]
    return top_scores, idx, counts


# speedup vs baseline: 1.8284x; 1.0409x over previous
import functools

import jax
import jax.numpy as jnp
from jax import lax
from jax.experimental import pallas as pl
from jax.experimental.pallas import tpu as pltpu, tpu_sc as plsc

N_TOKENS = 32768
DIM = 2048
NUM_EXPERTS = 8
TOP_K = 2
BLOCK_T = 2048

_SC_INFO = plsc.get_sparse_core_info()
_NC, _NS, _L = _SC_INFO.num_cores, _SC_INFO.num_subcores, _SC_INFO.num_lanes
_NW = _NC * _NS
_TOK_PER_W = N_TOKENS // _NW


def _gate_kernel(x_ref, wt_ref, scores_ref):
    logits = jnp.dot(x_ref[...], wt_ref[...], preferred_element_type=jnp.float32)
    scores_ref[...] = jax.nn.sigmoid(logits).T


def _gate_scores_t(x, gate_w_t):
    return pl.pallas_call(
        _gate_kernel,
        grid=(N_TOKENS // BLOCK_T,),
        in_specs=[
            pl.BlockSpec((BLOCK_T, DIM), lambda i: (i, 0)),
            pl.BlockSpec((DIM, NUM_EXPERTS), lambda i: (0, 0)),
        ],
        out_specs=pl.BlockSpec((NUM_EXPERTS, BLOCK_T), lambda i: (0, i)),
        out_shape=jax.ShapeDtypeStruct((NUM_EXPERTS, N_TOKENS), jnp.float32),
    )(x, gate_w_t)


def _route_body(scores_hbm, bias_hbm, s1_hbm, s2_hbm, i1_hbm, i2_hbm, pcnt_hbm,
                sc_v, bias_v, s1_v, s2_v, i1_v, i2_v, cnt_v):
    wid = lax.axis_index("s") * _NC + lax.axis_index("c")
    base = wid * _TOK_PER_W
    pltpu.sync_copy(scores_hbm.at[:, pl.ds(base, _TOK_PER_W)], sc_v)
    pltpu.sync_copy(bias_hbm, bias_v)

    bias_regs = [bias_v[e, :] for e in range(NUM_EXPERTS)]
    zero = jnp.zeros((_L,), jnp.float32)

    def chunk(j, cnts):
        t = j * _L
        s0 = sc_v[0, pl.ds(t, _L)]
        r1 = s0 + bias_regs[0]
        g1 = s0
        i1 = jnp.zeros((_L,), jnp.int32)
        r2 = jnp.full((_L,), -jnp.inf, jnp.float32)
        g2 = zero
        i2 = jnp.zeros((_L,), jnp.int32)
        for e in range(1, NUM_EXPERTS):
            s = sc_v[e, pl.ds(t, _L)]
            r = s + bias_regs[e]
            ei = jnp.full((_L,), e, jnp.int32)
            gt1 = r > r1
            gt2 = r > r2
            r2 = jnp.where(gt1, r1, jnp.where(gt2, r, r2))
            g2 = jnp.where(gt1, g1, jnp.where(gt2, s, g2))
            i2 = jnp.where(gt1, i1, jnp.where(gt2, ei, i2))
            r1 = jnp.where(gt1, r, r1)
            g1 = jnp.where(gt1, s, g1)
            i1 = jnp.where(gt1, ei, i1)
        denom = g1 + g2 + 1e-20
        s1_v[pl.ds(t, _L)] = g1 / denom
        s2_v[pl.ds(t, _L)] = g2 / denom
        i1_v[pl.ds(t, _L)] = i1
        i2_v[pl.ds(t, _L)] = i2
        new = []
        for e in range(NUM_EXPERTS):
            hits = (jnp.where(i1 == e, 1.0, 0.0) + jnp.where(i2 == e, 1.0, 0.0))
            new.append(cnts[e] + hits)
        return tuple(new)

    cnts = lax.fori_loop(0, _TOK_PER_W // _L, chunk,
                         tuple(zero for _ in range(NUM_EXPERTS)))

    lane = lax.iota(jnp.int32, _L)
    total = jnp.zeros((_L,), jnp.float32)
    for e in range(NUM_EXPERTS):
        ce = cnts[e]
        for k in (1, 2, 4, 8):
            perm = jnp.bitwise_xor(lane, k)
            ce = ce + lax.gather(
                ce, perm[:, None],
                dimension_numbers=lax.GatherDimensionNumbers(
                    offset_dims=(), collapsed_slice_dims=(0,),
                    start_index_map=(0,)),
                slice_sizes=(1,),
                mode=lax.GatherScatterMode.PROMISE_IN_BOUNDS)
        total = total + jnp.where(lane == e, ce, 0.0)
    cnt_v[...] = total

    pltpu.sync_copy(s1_v, s1_hbm.at[pl.ds(base, _TOK_PER_W)])
    pltpu.sync_copy(s2_v, s2_hbm.at[pl.ds(base, _TOK_PER_W)])
    pltpu.sync_copy(i1_v, i1_hbm.at[pl.ds(base, _TOK_PER_W)])
    pltpu.sync_copy(i2_v, i2_hbm.at[pl.ds(base, _TOK_PER_W)])
    pltpu.sync_copy(cnt_v, pcnt_hbm.at[wid])


_route = functools.partial(
    pl.kernel,
    mesh=plsc.VectorSubcoreMesh(core_axis_name="c", subcore_axis_name="s"),
    out_type=[
        jax.ShapeDtypeStruct((N_TOKENS,), jnp.float32),
        jax.ShapeDtypeStruct((N_TOKENS,), jnp.float32),
        jax.ShapeDtypeStruct((N_TOKENS,), jnp.int32),
        jax.ShapeDtypeStruct((N_TOKENS,), jnp.int32),
        jax.ShapeDtypeStruct((_NW, _L), jnp.float32),
    ],
    scratch_types=[
        pltpu.VMEM((NUM_EXPERTS, _TOK_PER_W), jnp.float32),
        pltpu.VMEM((NUM_EXPERTS, _L), jnp.float32),
        pltpu.VMEM((_TOK_PER_W,), jnp.float32),
        pltpu.VMEM((_TOK_PER_W,), jnp.float32),
        pltpu.VMEM((_TOK_PER_W,), jnp.int32),
        pltpu.VMEM((_TOK_PER_W,), jnp.int32),
        pltpu.VMEM((_L,), jnp.float32),
    ],
)(_route_body)


@jax.jit
def kernel(x, expert_bias, gate_w):
    scores_t = _gate_scores_t(x, gate_w.T)
    bias_b = jnp.broadcast_to(expert_bias[:, None], (NUM_EXPERTS, _L))
    s1, s2, i1, i2, pcnt = _route(scores_t, bias_b)
    top_scores = jnp.stack([s1, s2], axis=1)
    idx = jnp.stack([i1, i2], axis=1).astype(jnp.int64)
    counts = jnp.sum(pcnt, axis=0)[:NUM_EXPERTS]
    return top_scores, idx, counts
